# baseline (speedup 1.0000x reference)
import jax
import jax.numpy as jnp
from jax.experimental import pallas as pl
from jax.experimental.pallas import tpu as pltpu

_N = 4096
_HALF_ROWS = 2048
_FEAT = 32
_BM = 128
_NBUF = 3
_CHUNKS = [(0, 64), (64, 64)] + [(128 + i * _BM, _BM) for i in range((_N - 128) // _BM)]
_NCHUNK = len(_CHUNKS)


def _dot(a, e):
    return jnp.dot(a, e, preferred_element_type=jnp.float32)


def _body(a1_hbm, a2_hbm, e1_ref, e2_ref, w_ref, d_ref, p_ref, m_ref, b1, b2, sems):
    def copies(c):
        start, rows = _CHUNKS[c]
        slot = c % _NBUF
        return (
            pltpu.make_async_copy(
                a1_hbm.at[:, pl.ds(start, rows), :], b1.at[slot, :, pl.ds(0, rows), :],
                sems.at[slot, 0]),
            pltpu.make_async_copy(
                a2_hbm.at[:, pl.ds(start, rows), :], b2.at[slot, :, pl.ds(0, rows), :],
                sems.at[slot, 1]),
        )

    for c in range(_NBUF - 1):
        for cp in copies(c):
            cp.start()

    e1 = e1_ref[...]
    e2 = e2_ref[...]
    w = w_ref[0]

    for c in range(_NCHUNK):
        start, rows = _CHUNKS[c]
        slot = c % _NBUF
        cp1, cp2 = copies(c)
        cp1.wait()
        a1 = b1[slot, :, pl.ds(0, rows), :]
        t1 = jnp.maximum(_dot(a1[0], e1), 0.0) + jnp.maximum(_dot(a1[1], e1), 0.0)
        cp2.wait()
        a2 = b2[slot, :, pl.ds(0, rows), :]
        t2 = jnp.maximum(_dot(a2[0], e2), 0.0) + jnp.maximum(_dot(a2[1], e2), 0.0)
        if c + _NBUF - 1 < _NCHUNK:
            for cp in copies(c + _NBUF - 1):
                cp.start()
        t1 = t1 + t1
        t2 = t2 + t2
        if start < _HALF_ROWS:
            d_ref[pl.ds(start, rows), :] = t1
            p_ref[pl.ds(start, rows), :] = t2
        else:
            m_ref[pl.ds(start - _HALF_ROWS, rows), :] = w * t1 + (1.0 - w) * t2


def kernel(adj1, adj2, dEmbed, mEmbed, pEmbed, inter):
    e1 = jnp.concatenate([dEmbed, mEmbed], axis=0)
    e2 = jnp.concatenate([pEmbed, mEmbed], axis=0)
    d_out, p_out, m_out = pl.pallas_call(
        _body,
        in_specs=[
            pl.BlockSpec(memory_space=pl.ANY),
            pl.BlockSpec(memory_space=pl.ANY),
            pl.BlockSpec(memory_space=pltpu.VMEM),
            pl.BlockSpec(memory_space=pltpu.VMEM),
            pl.BlockSpec(memory_space=pltpu.SMEM),
        ],
        out_specs=[
            pl.BlockSpec(memory_space=pltpu.VMEM),
            pl.BlockSpec(memory_space=pltpu.VMEM),
            pl.BlockSpec(memory_space=pltpu.VMEM),
        ],
        out_shape=[
            jax.ShapeDtypeStruct((_HALF_ROWS, _FEAT), jnp.float32),
            jax.ShapeDtypeStruct((_HALF_ROWS, _FEAT), jnp.float32),
            jax.ShapeDtypeStruct((_HALF_ROWS, _FEAT), jnp.float32),
        ],
        scratch_shapes=[
            pltpu.VMEM((_NBUF, 2, _BM, _N), jnp.float32),
            pltpu.VMEM((_NBUF, 2, _BM, _N), jnp.float32),
            pltpu.SemaphoreType.DMA((_NBUF, 2)),
        ],
    )(adj1, adj2, e1, e2, inter)
    return (m_out, d_out, p_out)
